# initial kernel scaffold (unmeasured)
import jax
import jax.numpy as jnp
from jax import lax
from jax.experimental import pallas as pl
from jax.experimental.pallas import tpu as pltpu

N_DEV = 16


def kernel(x, w_mat):
    m_per, k = x.shape
    _, n = w_mat.shape
    n_per = n // N_DEV

    def body(x_ref, w_hbm, out_ref, w_tile, y_buf, w_sems, send_sems, recv_sems):
        my_i = lax.axis_index("i")

        barrier = pltpu.get_barrier_semaphore()
        for t in range(1, N_DEV):
            peer = lax.rem(my_i + t, N_DEV)
            pl.semaphore_signal(
                barrier, inc=1,
                device_id=(peer,), device_id_type=pl.DeviceIdType.MESH,
            )
        pl.semaphore_wait(barrier, N_DEV - 1)

        for t in range(N_DEV):
            j = lax.rem(my_i + t, N_DEV)
            slot = t % 2
            cp = pltpu.make_async_copy(
                w_hbm.at[:, pl.ds(j * n_per, n_per)],
                w_tile.at[slot],
                w_sems.at[slot],
            )
            cp.start()
            cp.wait()
            y = jnp.maximum(
                jnp.dot(x_ref[...], w_tile[slot],
                        preferred_element_type=jnp.float32),
                0.0,
            )
            if t == 0:
                out_ref[pl.ds(my_i * m_per, m_per), :] = y
            else:
                y_buf[slot] = y
                rdma = pltpu.make_async_remote_copy(
                    src_ref=y_buf.at[slot],
                    dst_ref=out_ref.at[pl.ds(my_i * m_per, m_per), :],
                    send_sem=send_sems.at[t],
                    recv_sem=recv_sems.at[t],
                    device_id=(j,),
                    device_id_type=pl.DeviceIdType.MESH,
                )
                rdma.start()
                rdma.wait_send()

        for t in range(1, N_DEV):
            src = lax.rem(my_i - t + N_DEV, N_DEV)
            recv = pltpu.make_async_remote_copy(
                src_ref=y_buf.at[0],
                dst_ref=out_ref.at[pl.ds(src * m_per, m_per), :],
                send_sem=send_sems.at[t],
                recv_sem=recv_sems.at[t],
                device_id=(my_i,),
                device_id_type=pl.DeviceIdType.MESH,
            )
            recv.wait_recv()

    return pl.pallas_call(
        body,
        out_shape=jax.ShapeDtypeStruct((N_DEV * m_per, n_per), jnp.float32),
        in_specs=[
            pl.BlockSpec(memory_space=pltpu.VMEM),
            pl.BlockSpec(memory_space=pltpu.ANY),
        ],
        out_specs=pl.BlockSpec(memory_space=pltpu.VMEM),
        scratch_shapes=[
            pltpu.VMEM((2, k, n_per), jnp.float32),
            pltpu.VMEM((2, m_per, n_per), jnp.float32),
            pltpu.SemaphoreType.DMA((2,)),
            pltpu.SemaphoreType.DMA((N_DEV,)),
            pltpu.SemaphoreType.DMA((N_DEV,)),
        ],
        compiler_params=pltpu.CompilerParams(collective_id=0),
    )(x, w_mat)


# baseline (device time: 250084 ns/iter reference)
import jax
import jax.numpy as jnp
from jax import lax
from jax.experimental import pallas as pl
from jax.experimental.pallas import tpu as pltpu

N_DEV = 16


def kernel(x, w_mat):
    m_per, k = x.shape
    _, n = w_mat.shape
    n_per = n // N_DEV

    def body(x_ref, w_hbm, out_ref, w_tile, y_buf, w_sems, send_sems, recv_sems):
        my_i = lax.axis_index("i")

        barrier = pltpu.get_barrier_semaphore()
        for t in range(1, N_DEV):
            peer = lax.rem(my_i + t, N_DEV)
            pl.semaphore_signal(
                barrier, inc=1,
                device_id=(peer,), device_id_type=pl.DeviceIdType.MESH,
            )
        pl.semaphore_wait(barrier, N_DEV - 1)

        for t in range(N_DEV):
            j = lax.rem(my_i + t, N_DEV)
            slot = t % 2
            cp = pltpu.make_async_copy(
                w_hbm.at[:, pl.ds(j * n_per, n_per)],
                w_tile.at[slot],
                w_sems.at[slot],
            )
            cp.start()
            cp.wait()
            y = jnp.maximum(
                jnp.dot(x_ref[...], w_tile[slot],
                        preferred_element_type=jnp.float32),
                0.0,
            )
            if t == 0:
                out_ref[pl.ds(my_i * m_per, m_per), :] = y
            else:
                y_buf[slot] = y
                rdma = pltpu.make_async_remote_copy(
                    src_ref=y_buf.at[slot],
                    dst_ref=out_ref.at[pl.ds(my_i * m_per, m_per), :],
                    send_sem=send_sems.at[t],
                    recv_sem=recv_sems.at[t],
                    device_id=(j,),
                    device_id_type=pl.DeviceIdType.MESH,
                )
                rdma.start()
                rdma.wait_send()

        for t in range(1, N_DEV):
            src = lax.rem(my_i - t + N_DEV, N_DEV)
            recv = pltpu.make_async_remote_copy(
                src_ref=y_buf.at[0],
                dst_ref=out_ref.at[pl.ds(src * m_per, m_per), :],
                send_sem=send_sems.at[t],
                recv_sem=recv_sems.at[t],
                device_id=(my_i,),
                device_id_type=pl.DeviceIdType.MESH,
            )
            recv.wait_recv()

    return pl.pallas_call(
        body,
        out_shape=jax.ShapeDtypeStruct((N_DEV * m_per, n_per), jnp.float32),
        in_specs=[
            pl.BlockSpec(memory_space=pltpu.VMEM),
            pl.BlockSpec(memory_space=pl.ANY),
        ],
        out_specs=pl.BlockSpec(memory_space=pltpu.VMEM),
        scratch_shapes=[
            pltpu.VMEM((2, k, n_per), jnp.float32),
            pltpu.VMEM((2, m_per, n_per), jnp.float32),
            pltpu.SemaphoreType.DMA((2,)),
            pltpu.SemaphoreType.DMA((N_DEV,)),
            pltpu.SemaphoreType.DMA((N_DEV,)),
        ],
        compiler_params=pltpu.CompilerParams(
            collective_id=0, vmem_limit_bytes=100 * 1024 * 1024
        ),
    )(x, w_mat)


# device time: 130625 ns/iter; 1.9145x vs baseline; 1.9145x over previous
import jax
import jax.numpy as jnp
from jax import lax
from jax.experimental import pallas as pl
from jax.experimental.pallas import tpu as pltpu

N_DEV = 16


def kernel(x, w_mat):
    m_per, k = x.shape
    _, n = w_mat.shape
    n_per = n // N_DEV

    def body(x_ref, w_hbm, out_ref, w_tile, y_stage, w_sems, send_sems, recv_sems):
        my_i = lax.axis_index("i")

        barrier = pltpu.get_barrier_semaphore()
        for t in range(1, N_DEV):
            peer = lax.rem(my_i + t, N_DEV)
            pl.semaphore_signal(
                barrier, inc=1,
                device_id=(peer,), device_id_type=pl.DeviceIdType.MESH,
            )
        pl.semaphore_wait(barrier, N_DEV - 1)

        def w_dma(t, slot):
            j = lax.rem(my_i + t, N_DEV)
            return pltpu.make_async_copy(
                w_hbm.at[:, pl.ds(j * n_per, n_per)],
                w_tile.at[slot],
                w_sems.at[slot],
            )

        w_dma(0, 0).start()
        sends = []
        for t in range(N_DEV):
            slot = t % 2
            if t + 1 < N_DEV:
                w_dma(t + 1, (t + 1) % 2).start()
            w_dma(t, slot).wait()
            j = lax.rem(my_i + t, N_DEV)
            y = jnp.maximum(
                jnp.dot(x_ref[...], w_tile[slot],
                        preferred_element_type=jnp.float32),
                0.0,
            )
            if t == 0:
                out_ref[pl.ds(my_i * m_per, m_per), :] = y
            else:
                y_stage[:, t * n_per:(t + 1) * n_per] = y
                rdma = pltpu.make_async_remote_copy(
                    src_ref=y_stage.at[:, pl.ds(t * n_per, n_per)],
                    dst_ref=out_ref.at[pl.ds(my_i * m_per, m_per), :],
                    send_sem=send_sems.at[t],
                    recv_sem=recv_sems.at[t],
                    device_id=(j,),
                    device_id_type=pl.DeviceIdType.MESH,
                )
                rdma.start()
                sends.append(rdma)

        for rdma in sends:
            rdma.wait_send()

        for t in range(1, N_DEV):
            src = lax.rem(my_i - t + N_DEV, N_DEV)
            recv = pltpu.make_async_remote_copy(
                src_ref=y_stage.at[:, pl.ds(t * n_per, n_per)],
                dst_ref=out_ref.at[pl.ds(src * m_per, m_per), :],
                send_sem=send_sems.at[t],
                recv_sem=recv_sems.at[t],
                device_id=(my_i,),
                device_id_type=pl.DeviceIdType.MESH,
            )
            recv.wait_recv()

    return pl.pallas_call(
        body,
        out_shape=jax.ShapeDtypeStruct((N_DEV * m_per, n_per), jnp.float32),
        in_specs=[
            pl.BlockSpec(memory_space=pltpu.VMEM),
            pl.BlockSpec(memory_space=pl.ANY),
        ],
        out_specs=pl.BlockSpec(memory_space=pltpu.VMEM),
        scratch_shapes=[
            pltpu.VMEM((2, k, n_per), jnp.float32),
            pltpu.VMEM((m_per, n), jnp.float32),
            pltpu.SemaphoreType.DMA((2,)),
            pltpu.SemaphoreType.DMA((N_DEV,)),
            pltpu.SemaphoreType.DMA((N_DEV,)),
        ],
        compiler_params=pltpu.CompilerParams(
            collective_id=0, vmem_limit_bytes=100 * 1024 * 1024
        ),
    )(x, w_mat)
